# baseline (speedup 1.0000x reference)
import jax
import jax.numpy as jnp
from jax import lax
from jax.experimental import pallas as pl
from jax.experimental.pallas import tpu as pltpu
from jax.experimental.pallas import tpu_sc as plsc

N_NODES = 10000
N_PAD = 10240
N_EDGES = 320000
D = 128
NC = 2
NS = 16
E_PER_W = N_EDGES // (NC * NS)
DCH = 80
DNCH = E_PER_W // DCH
CHUNK = 40
NCHUNK = E_PER_W // CHUNK
NBUF = 5
NGRP = NCHUNK // NBUF
ROWB = 400

_mesh = plsc.VectorSubcoreMesh(core_axis_name="c", subcore_axis_name="s")


def _deg_body(e4_hbm, out_hbm, idxs_v, idxd_v, ones_v, zeros_v,
              acc_s, acc_d, dsem):
    c = lax.axis_index("c")
    s = lax.axis_index("s")
    w = c * NS + s

    @pl.loop(0, DCH, step=16)
    def _(i):
        ones_v[pl.ds(i, 16)] = jnp.ones((16,), jnp.float32)

    @pl.loop(0, 640, step=16)
    def _(i):
        zeros_v[pl.ds(i, 16)] = jnp.zeros((16,), jnp.float32)

    pltpu.sync_copy(zeros_v, acc_s.at[pl.ds(s * 640, 640)])
    pltpu.sync_copy(zeros_v, acc_d.at[pl.ds(s * 640, 640)])
    plsc.subcore_barrier()

    pltpu.sync_copy(e4_hbm.at[0, w], idxs_v)
    pltpu.sync_copy(e4_hbm.at[1, w], idxd_v)

    @pl.loop(0, DNCH)
    def _(k):
        pltpu.sync_copy(ones_v, acc_s.at[idxs_v.at[k]], add=True)
        pltpu.sync_copy(ones_v, acc_d.at[idxd_v.at[k]], add=True)

    plsc.subcore_barrier()

    pltpu.sync_copy(acc_s.at[pl.ds(s * 640, 640)],
                    out_hbm.at[c, 0, pl.ds(s * 640, 640)])
    pltpu.sync_copy(acc_d.at[pl.ds(s * 640, 640)],
                    out_hbm.at[c, 1, pl.ds(s * 640, 640)])


def _deg_kernel(e4):
    return pl.kernel(
        _deg_body,
        out_type=jax.ShapeDtypeStruct((NC, 2, N_PAD), jnp.float32),
        mesh=_mesh,
        scratch_types=[
            pltpu.VMEM((DNCH, DCH), jnp.int32),
            pltpu.VMEM((DNCH, DCH), jnp.int32),
            pltpu.VMEM((DCH,), jnp.float32),
            pltpu.VMEM((640,), jnp.float32),
            pltpu.VMEM_SHARED((N_PAD,), jnp.float32),
            pltpu.VMEM_SHARED((N_PAD,), jnp.float32),
            pltpu.SemaphoreType.DMA((2,)),
        ],
    )(e4)


def _mm_body(x_ref, w_ref, z_ref):
    z_ref[...] = jnp.dot(x_ref[...], w_ref[...],
                         preferred_element_type=jnp.float32,
                         precision=lax.Precision.HIGHEST)


def _mm_kernel(x, W):
    return pl.pallas_call(
        _mm_body,
        grid=(N_NODES // ROWB,),
        in_specs=[
            pl.BlockSpec((ROWB, D), lambda i: (i, 0)),
            pl.BlockSpec((D, D), lambda i: (0, 0)),
        ],
        out_specs=pl.BlockSpec((ROWB, D), lambda i: (i, 0)),
        out_shape=jax.ShapeDtypeStruct((N_NODES, D), jnp.float32),
    )(x, W)


def _scale_body(x_ref, deg_ref, y_ref):
    d = deg_ref[0, 0, 0, 0, :] + deg_ref[1, 0, 0, 0, :]
    norm = lax.rsqrt(jnp.clip(d, 1.0, None))
    y_ref[...] = x_ref[...] * norm[:, None]


def _scale_kernel(x, degp):
    return pl.pallas_call(
        _scale_body,
        grid=(N_NODES // ROWB,),
        in_specs=[
            pl.BlockSpec((ROWB, D), lambda i: (i, 0)),
            pl.BlockSpec((NC, 2, 1, 1, ROWB), lambda i: (0, 0, i, 0, 0)),
        ],
        out_specs=pl.BlockSpec((ROWB, D), lambda i: (i, 0)),
        out_shape=jax.ShapeDtypeStruct((N_NODES, D), jnp.float32),
    )(x, degp)


def _edge_body(y_hbm, e4_hbm, out_hbm, idxs_v, idxd_v, rows_v, acc,
               gsem, ssem, isem):
    c = lax.axis_index("c")
    s = lax.axis_index("s")
    w = c * NS + s

    @pl.loop(0, CHUNK)
    def _(r):
        @pl.loop(0, D, step=16)
        def _(j):
            rows_v[0, r, pl.ds(j, 16)] = jnp.zeros((16,), jnp.float32)

    @pl.loop(0, 640 // CHUNK)
    def _(k):
        pltpu.sync_copy(rows_v.at[0], acc.at[pl.ds(s * 640 + k * CHUNK, CHUNK)])

    plsc.subcore_barrier()

    for q in range(3):
        pltpu.sync_copy(e4_hbm.at[0, w, q], idxs_v.at[q])
        pltpu.sync_copy(e4_hbm.at[1, w, q], idxd_v.at[q])
    for b in range(NBUF - 1):
        pltpu.async_copy(y_hbm.at[idxs_v.at[0, b]], rows_v.at[b], gsem.at[b])

    def visit(q, qn, b, skip_swait, skip_gfire):
        bp = (b + NBUF - 1) % NBUF
        if not skip_swait:
            pltpu.make_async_copy(y_hbm.at[pl.ds(0, CHUNK)], rows_v.at[bp],
                                  ssem).wait()
        if not skip_gfire:
            if b == 0:
                src_idx = idxs_v.at[q, NBUF - 1]
            else:
                src_idx = idxs_v.at[qn, b - 1]
            pltpu.async_copy(y_hbm.at[src_idx], rows_v.at[bp], gsem.at[bp])
        pltpu.make_async_copy(y_hbm.at[pl.ds(0, CHUNK)], rows_v.at[b],
                              gsem.at[b]).wait()
        pltpu.async_copy(rows_v.at[b], acc.at[idxd_v.at[q, b]], ssem,
                         add=True)

    def group(e, q, wait_idx, fire_load, first=False, last=False):
        if wait_idx:
            pltpu.make_async_copy(e4_hbm.at[0, 0, 0], idxs_v.at[q],
                                  isem).wait()
            pltpu.make_async_copy(e4_hbm.at[0, 0, 0], idxd_v.at[q],
                                  isem).wait()
        qn = (q + 1) % 3
        visit(q, qn, 0, skip_swait=first, skip_gfire=False)
        if fire_load:
            qp = (q + 2) % 3
            pltpu.async_copy(e4_hbm.at[0, w, e + 2], idxs_v.at[qp], isem)
            pltpu.async_copy(e4_hbm.at[1, w, e + 2], idxd_v.at[qp], isem)
        for b in range(1, NBUF):
            visit(q, qn, b, skip_swait=False, skip_gfire=(last and b >= 1))

    group(0, 0, wait_idx=False, fire_load=False, first=True)
    group(1, 1, wait_idx=False, fire_load=True)

    @pl.loop(2, NGRP - 3, step=3)
    def _(g):
        for i, q in enumerate((2, 0, 1)):
            group(g + i, q, wait_idx=True, fire_load=True)

    group(47, 2, wait_idx=True, fire_load=True)
    group(48, 0, wait_idx=True, fire_load=False)
    group(49, 1, wait_idx=False, fire_load=False, last=True)

    pltpu.make_async_copy(y_hbm.at[pl.ds(0, CHUNK)], rows_v.at[NBUF - 1],
                          ssem).wait()

    plsc.subcore_barrier()
    pltpu.sync_copy(acc.at[pl.ds(s * 640, 640)],
                    out_hbm.at[c, pl.ds(s * 640, 640)])


def _edge_kernel(y, e4):
    return pl.kernel(
        _edge_body,
        out_type=jax.ShapeDtypeStruct((NC, N_PAD, D), jnp.float32),
        mesh=_mesh,
        scratch_types=[
            pltpu.VMEM((3, NBUF, CHUNK), jnp.int32),
            pltpu.VMEM((3, NBUF, CHUNK), jnp.int32),
            pltpu.VMEM((NBUF, CHUNK, D), jnp.float32),
            pltpu.VMEM_SHARED((N_PAD, D), jnp.float32),
            pltpu.SemaphoreType.DMA((NBUF,)),
            pltpu.SemaphoreType.DMA,
            pltpu.SemaphoreType.DMA,
        ],
    )(y, e4)


def _out_body(p_ref, deg_ref, b_ref, o_ref):
    agg = p_ref[0] + p_ref[1]
    d = deg_ref[0, 1, 0, 0, :] + deg_ref[1, 1, 0, 0, :]
    norm = lax.rsqrt(jnp.clip(d, 1.0, None))
    o_ref[...] = jnp.maximum(agg * norm[:, None] + b_ref[...], 0.0)


def _out_kernel(p, degp, b2):
    return pl.pallas_call(
        _out_body,
        grid=(N_NODES // ROWB,),
        in_specs=[
            pl.BlockSpec((NC, ROWB, D), lambda i: (0, i, 0)),
            pl.BlockSpec((NC, 2, 1, 1, ROWB), lambda i: (0, 0, i, 0, 0)),
            pl.BlockSpec((1, D), lambda i: (0, 0)),
        ],
        out_specs=pl.BlockSpec((ROWB, D), lambda i: (i, 0)),
        out_shape=jax.ShapeDtypeStruct((N_NODES, D), jnp.float32),
    )(p, degp, b2)


def kernel(x, edge_index, W, b):
    e4 = edge_index.reshape(2, NC * NS, DNCH, DCH)
    e5 = edge_index.reshape(2, NC * NS, NGRP, NBUF, CHUNK)
    z = _mm_kernel(x, W)
    degp = _deg_kernel(e4)
    degt = degp[:, :, :N_NODES].reshape(NC, 2, N_NODES // ROWB, 1, ROWB)
    y = _scale_kernel(z, degt)
    p = _edge_kernel(y, e5)
    return _out_kernel(p, degt, b.reshape(1, D))


# speedup vs baseline: 1.0060x; 1.0060x over previous
import jax
import jax.numpy as jnp
from jax import lax
from jax.experimental import pallas as pl
from jax.experimental.pallas import tpu as pltpu
from jax.experimental.pallas import tpu_sc as plsc

N_NODES = 10000
N_PAD = 10240
N_EDGES = 320000
D = 128
NC = 2
NS = 16
E_PER_W = N_EDGES // (NC * NS)
DCH = 80
DNCH = E_PER_W // DCH
CHUNK = 40
NCHUNK = E_PER_W // CHUNK
NBUF = 5
NGRP = NCHUNK // NBUF
ROWB = 400

_mesh = plsc.VectorSubcoreMesh(core_axis_name="c", subcore_axis_name="s")


def _deg_body(e4_hbm, out_hbm, idxs_v, idxd_v, ones_v, zeros_v,
              acc_s, acc_d, dsem):
    c = lax.axis_index("c")
    s = lax.axis_index("s")
    w = c * NS + s

    @pl.loop(0, DCH, step=16)
    def _(i):
        ones_v[pl.ds(i, 16)] = jnp.ones((16,), jnp.float32)

    @pl.loop(0, 640, step=16)
    def _(i):
        zeros_v[pl.ds(i, 16)] = jnp.zeros((16,), jnp.float32)

    pltpu.sync_copy(zeros_v, acc_s.at[pl.ds(s * 640, 640)])
    pltpu.sync_copy(zeros_v, acc_d.at[pl.ds(s * 640, 640)])
    plsc.subcore_barrier()

    pltpu.sync_copy(e4_hbm.at[0, w], idxs_v)
    pltpu.sync_copy(e4_hbm.at[1, w], idxd_v)

    @pl.loop(0, DNCH)
    def _(k):
        pltpu.sync_copy(ones_v, acc_s.at[idxs_v.at[k]], add=True)
        pltpu.sync_copy(ones_v, acc_d.at[idxd_v.at[k]], add=True)

    plsc.subcore_barrier()

    pltpu.sync_copy(acc_s.at[pl.ds(s * 640, 640)],
                    out_hbm.at[c, 0, pl.ds(s * 640, 640)])
    pltpu.sync_copy(acc_d.at[pl.ds(s * 640, 640)],
                    out_hbm.at[c, 1, pl.ds(s * 640, 640)])


def _deg_kernel(e4):
    return pl.kernel(
        _deg_body,
        out_type=jax.ShapeDtypeStruct((NC, 2, N_PAD), jnp.float32),
        mesh=_mesh,
        scratch_types=[
            pltpu.VMEM((DNCH, DCH), jnp.int32),
            pltpu.VMEM((DNCH, DCH), jnp.int32),
            pltpu.VMEM((DCH,), jnp.float32),
            pltpu.VMEM((640,), jnp.float32),
            pltpu.VMEM_SHARED((N_PAD,), jnp.float32),
            pltpu.VMEM_SHARED((N_PAD,), jnp.float32),
            pltpu.SemaphoreType.DMA((2,)),
        ],
    )(e4)


def _scale_body(x_ref, w_ref, deg_ref, y_ref):
    d = deg_ref[0, 0, 0, 0, :] + deg_ref[1, 0, 0, 0, :]
    norm = lax.rsqrt(jnp.clip(d, 1.0, None))
    z = jnp.dot(x_ref[...], w_ref[...], preferred_element_type=jnp.float32,
                precision=lax.Precision.HIGHEST)
    y_ref[...] = z * norm[:, None]


def _scale_kernel(x, W, degp):
    return pl.pallas_call(
        _scale_body,
        grid=(N_NODES // ROWB,),
        in_specs=[
            pl.BlockSpec((ROWB, D), lambda i: (i, 0)),
            pl.BlockSpec((D, D), lambda i: (0, 0)),
            pl.BlockSpec((NC, 2, 1, 1, ROWB), lambda i: (0, 0, i, 0, 0)),
        ],
        out_specs=pl.BlockSpec((ROWB, D), lambda i: (i, 0)),
        out_shape=jax.ShapeDtypeStruct((N_NODES, D), jnp.float32),
    )(x, W, degp)


def _edge_body(y_hbm, e4_hbm, out_hbm, idxs_v, idxd_v, rows_v, acc,
               gsem, ssem, isem):
    c = lax.axis_index("c")
    s = lax.axis_index("s")
    w = c * NS + s

    @pl.loop(0, CHUNK)
    def _(r):
        @pl.loop(0, D, step=16)
        def _(j):
            rows_v[0, r, pl.ds(j, 16)] = jnp.zeros((16,), jnp.float32)

    @pl.loop(0, 640 // CHUNK)
    def _(k):
        pltpu.sync_copy(rows_v.at[0], acc.at[pl.ds(s * 640 + k * CHUNK, CHUNK)])

    plsc.subcore_barrier()

    for q in range(3):
        pltpu.sync_copy(e4_hbm.at[0, w, q], idxs_v.at[q])
        pltpu.sync_copy(e4_hbm.at[1, w, q], idxd_v.at[q])
    for b in range(NBUF - 1):
        pltpu.async_copy(y_hbm.at[idxs_v.at[0, b]], rows_v.at[b], gsem.at[b])

    def visit(q, qn, b, skip_swait, skip_gfire):
        bp = (b + NBUF - 1) % NBUF
        if not skip_swait:
            pltpu.make_async_copy(y_hbm.at[pl.ds(0, CHUNK)], rows_v.at[bp],
                                  ssem).wait()
        if not skip_gfire:
            if b == 0:
                src_idx = idxs_v.at[q, NBUF - 1]
            else:
                src_idx = idxs_v.at[qn, b - 1]
            pltpu.async_copy(y_hbm.at[src_idx], rows_v.at[bp], gsem.at[bp])
        pltpu.make_async_copy(y_hbm.at[pl.ds(0, CHUNK)], rows_v.at[b],
                              gsem.at[b]).wait()
        pltpu.async_copy(rows_v.at[b], acc.at[idxd_v.at[q, b]], ssem,
                         add=True)

    def group(e, q, wait_idx, fire_load, first=False, last=False):
        if wait_idx:
            pltpu.make_async_copy(e4_hbm.at[0, 0, 0], idxs_v.at[q],
                                  isem).wait()
            pltpu.make_async_copy(e4_hbm.at[0, 0, 0], idxd_v.at[q],
                                  isem).wait()
        qn = (q + 1) % 3
        visit(q, qn, 0, skip_swait=first, skip_gfire=False)
        if fire_load:
            qp = (q + 2) % 3
            pltpu.async_copy(e4_hbm.at[0, w, e + 2], idxs_v.at[qp], isem)
            pltpu.async_copy(e4_hbm.at[1, w, e + 2], idxd_v.at[qp], isem)
        for b in range(1, NBUF):
            visit(q, qn, b, skip_swait=False, skip_gfire=(last and b >= 1))

    group(0, 0, wait_idx=False, fire_load=False, first=True)
    group(1, 1, wait_idx=False, fire_load=True)

    @pl.loop(2, NGRP - 3, step=3)
    def _(g):
        for i, q in enumerate((2, 0, 1)):
            group(g + i, q, wait_idx=True, fire_load=True)

    group(47, 2, wait_idx=True, fire_load=True)
    group(48, 0, wait_idx=True, fire_load=False)
    group(49, 1, wait_idx=False, fire_load=False, last=True)

    pltpu.make_async_copy(y_hbm.at[pl.ds(0, CHUNK)], rows_v.at[NBUF - 1],
                          ssem).wait()

    plsc.subcore_barrier()
    pltpu.sync_copy(acc.at[pl.ds(s * 640, 640)],
                    out_hbm.at[c, pl.ds(s * 640, 640)])


def _edge_kernel(y, e4):
    return pl.kernel(
        _edge_body,
        out_type=jax.ShapeDtypeStruct((NC, N_PAD, D), jnp.float32),
        mesh=_mesh,
        scratch_types=[
            pltpu.VMEM((3, NBUF, CHUNK), jnp.int32),
            pltpu.VMEM((3, NBUF, CHUNK), jnp.int32),
            pltpu.VMEM((NBUF, CHUNK, D), jnp.float32),
            pltpu.VMEM_SHARED((N_PAD, D), jnp.float32),
            pltpu.SemaphoreType.DMA((NBUF,)),
            pltpu.SemaphoreType.DMA,
            pltpu.SemaphoreType.DMA,
        ],
    )(y, e4)


def _out_body(p_ref, deg_ref, b_ref, o_ref):
    agg = p_ref[0] + p_ref[1]
    d = deg_ref[0, 1, 0, 0, :] + deg_ref[1, 1, 0, 0, :]
    norm = lax.rsqrt(jnp.clip(d, 1.0, None))
    o_ref[...] = jnp.maximum(agg * norm[:, None] + b_ref[...], 0.0)


def _out_kernel(p, degp, b2):
    return pl.pallas_call(
        _out_body,
        grid=(N_NODES // ROWB,),
        in_specs=[
            pl.BlockSpec((NC, ROWB, D), lambda i: (0, i, 0)),
            pl.BlockSpec((NC, 2, 1, 1, ROWB), lambda i: (0, 0, i, 0, 0)),
            pl.BlockSpec((1, D), lambda i: (0, 0)),
        ],
        out_specs=pl.BlockSpec((ROWB, D), lambda i: (i, 0)),
        out_shape=jax.ShapeDtypeStruct((N_NODES, D), jnp.float32),
    )(p, degp, b2)


def kernel(x, edge_index, W, b):
    e4 = edge_index.reshape(2, NC * NS, DNCH, DCH)
    e5 = edge_index.reshape(2, NC * NS, NGRP, NBUF, CHUNK)
    degp = _deg_kernel(e4)
    degt = degp[:, :, :N_NODES].reshape(NC, 2, N_NODES // ROWB, 1, ROWB)
    y = _scale_kernel(x, W, degt)
    p = _edge_kernel(y, e5)
    return _out_kernel(p, degt, b.reshape(1, D))


# speedup vs baseline: 1.0392x; 1.0330x over previous
import jax
import jax.numpy as jnp
from jax import lax
from jax.experimental import pallas as pl
from jax.experimental.pallas import tpu as pltpu
from jax.experimental.pallas import tpu_sc as plsc

N_NODES = 10000
N_PAD = 10240
N_EDGES = 320000
D = 128
NC = 2
NS = 16
E_PER_W = N_EDGES // (NC * NS)
DCH = 128
DNCH = E_PER_W // DCH
CHUNK = 40
NCHUNK = E_PER_W // CHUNK
NBUF = 5
NGRP = NCHUNK // NBUF
ROWB = 400

_mesh = plsc.VectorSubcoreMesh(core_axis_name="c", subcore_axis_name="s")


def _deg_body(e4_hbm, et_hbm, out_hbm, idxs_v, idxd_v, idxt_v, ones_v,
              zeros_v, acc_s, acc_d, dsem):
    c = lax.axis_index("c")
    s = lax.axis_index("s")
    w = c * NS + s

    @pl.loop(0, DCH, step=16)
    def _(i):
        ones_v[pl.ds(i, 16)] = jnp.ones((16,), jnp.float32)

    @pl.loop(0, 640, step=16)
    def _(i):
        zeros_v[pl.ds(i, 16)] = jnp.zeros((16,), jnp.float32)

    pltpu.sync_copy(zeros_v, acc_s.at[pl.ds(s * 640, 640)])
    pltpu.sync_copy(zeros_v, acc_d.at[pl.ds(s * 640, 640)])
    plsc.subcore_barrier()

    pltpu.sync_copy(e4_hbm.at[0, w], idxs_v)
    pltpu.sync_copy(e4_hbm.at[1, w], idxd_v)
    pltpu.sync_copy(et_hbm.at[0, w], idxt_v.at[0])
    pltpu.sync_copy(et_hbm.at[1, w], idxt_v.at[1])

    @pl.loop(0, DNCH)
    def _(k):
        pltpu.sync_copy(ones_v, acc_s.at[idxs_v.at[k]], add=True)
        pltpu.sync_copy(ones_v, acc_d.at[idxd_v.at[k]], add=True)

    pltpu.sync_copy(ones_v.at[pl.ds(0, 16)], acc_s.at[idxt_v.at[0]], add=True)
    pltpu.sync_copy(ones_v.at[pl.ds(0, 16)], acc_d.at[idxt_v.at[1]], add=True)

    plsc.subcore_barrier()

    pltpu.sync_copy(acc_s.at[pl.ds(s * 640, 640)],
                    out_hbm.at[c, 0, pl.ds(s * 640, 640)])
    pltpu.sync_copy(acc_d.at[pl.ds(s * 640, 640)],
                    out_hbm.at[c, 1, pl.ds(s * 640, 640)])


def _deg_kernel(e4, et):
    return pl.kernel(
        _deg_body,
        out_type=jax.ShapeDtypeStruct((NC, 2, N_PAD), jnp.float32),
        mesh=_mesh,
        scratch_types=[
            pltpu.VMEM((DNCH, DCH), jnp.int32),
            pltpu.VMEM((DNCH, DCH), jnp.int32),
            pltpu.VMEM((2, 16), jnp.int32),
            pltpu.VMEM((DCH,), jnp.float32),
            pltpu.VMEM((640,), jnp.float32),
            pltpu.VMEM_SHARED((N_PAD,), jnp.float32),
            pltpu.VMEM_SHARED((N_PAD,), jnp.float32),
            pltpu.SemaphoreType.DMA((2,)),
        ],
    )(e4, et)


def _scale_body(x_ref, w_ref, deg_ref, y_ref):
    d = deg_ref[0, 0, 0, 0, :] + deg_ref[1, 0, 0, 0, :]
    norm = lax.rsqrt(jnp.clip(d, 1.0, None))
    z = jnp.dot(x_ref[...], w_ref[...], preferred_element_type=jnp.float32,
                precision=lax.Precision.HIGHEST)
    y_ref[...] = z * norm[:, None]


def _scale_kernel(x, W, degp):
    return pl.pallas_call(
        _scale_body,
        grid=(N_NODES // ROWB,),
        in_specs=[
            pl.BlockSpec((ROWB, D), lambda i: (i, 0)),
            pl.BlockSpec((D, D), lambda i: (0, 0)),
            pl.BlockSpec((NC, 2, 1, 1, ROWB), lambda i: (0, 0, i, 0, 0)),
        ],
        out_specs=pl.BlockSpec((ROWB, D), lambda i: (i, 0)),
        out_shape=jax.ShapeDtypeStruct((N_NODES, D), jnp.float32),
    )(x, W, degp)


def _edge_body(y_hbm, e4_hbm, out_hbm, idxs_v, idxd_v, rows_v, acc,
               gsem, ssem, isem):
    c = lax.axis_index("c")
    s = lax.axis_index("s")
    w = c * NS + s

    @pl.loop(0, CHUNK)
    def _(r):
        @pl.loop(0, D, step=16)
        def _(j):
            rows_v[0, r, pl.ds(j, 16)] = jnp.zeros((16,), jnp.float32)

    @pl.loop(0, 640 // CHUNK)
    def _(k):
        pltpu.sync_copy(rows_v.at[0], acc.at[pl.ds(s * 640 + k * CHUNK, CHUNK)])

    plsc.subcore_barrier()

    for q in range(3):
        pltpu.sync_copy(e4_hbm.at[0, w, q], idxs_v.at[q])
        pltpu.sync_copy(e4_hbm.at[1, w, q], idxd_v.at[q])
    for b in range(NBUF - 1):
        pltpu.async_copy(y_hbm.at[idxs_v.at[0, b]], rows_v.at[b], gsem.at[b])

    def visit(q, qn, b, skip_swait, skip_gfire):
        bp = (b + NBUF - 1) % NBUF
        if not skip_swait:
            pltpu.make_async_copy(y_hbm.at[pl.ds(0, CHUNK)], rows_v.at[bp],
                                  ssem).wait()
        if not skip_gfire:
            if b == 0:
                src_idx = idxs_v.at[q, NBUF - 1]
            else:
                src_idx = idxs_v.at[qn, b - 1]
            pltpu.async_copy(y_hbm.at[src_idx], rows_v.at[bp], gsem.at[bp])
        pltpu.make_async_copy(y_hbm.at[pl.ds(0, CHUNK)], rows_v.at[b],
                              gsem.at[b]).wait()
        pltpu.async_copy(rows_v.at[b], acc.at[idxd_v.at[q, b]], ssem,
                         add=True)

    def group(e, q, wait_idx, fire_load, first=False, last=False):
        if wait_idx:
            pltpu.make_async_copy(e4_hbm.at[0, 0, 0], idxs_v.at[q],
                                  isem).wait()
            pltpu.make_async_copy(e4_hbm.at[0, 0, 0], idxd_v.at[q],
                                  isem).wait()
        qn = (q + 1) % 3
        visit(q, qn, 0, skip_swait=first, skip_gfire=False)
        if fire_load:
            qp = (q + 2) % 3
            pltpu.async_copy(e4_hbm.at[0, w, e + 2], idxs_v.at[qp], isem)
            pltpu.async_copy(e4_hbm.at[1, w, e + 2], idxd_v.at[qp], isem)
        for b in range(1, NBUF):
            visit(q, qn, b, skip_swait=False, skip_gfire=(last and b >= 1))

    group(0, 0, wait_idx=False, fire_load=False, first=True)
    group(1, 1, wait_idx=False, fire_load=True)

    @pl.loop(2, NGRP - 3, step=3)
    def _(g):
        for i, q in enumerate((2, 0, 1)):
            group(g + i, q, wait_idx=True, fire_load=True)

    group(47, 2, wait_idx=True, fire_load=True)
    group(48, 0, wait_idx=True, fire_load=False)
    group(49, 1, wait_idx=False, fire_load=False, last=True)

    pltpu.make_async_copy(y_hbm.at[pl.ds(0, CHUNK)], rows_v.at[NBUF - 1],
                          ssem).wait()

    plsc.subcore_barrier()
    pltpu.sync_copy(acc.at[pl.ds(s * 640, 640)],
                    out_hbm.at[c, pl.ds(s * 640, 640)])


def _edge_kernel(y, e4):
    return pl.kernel(
        _edge_body,
        out_type=jax.ShapeDtypeStruct((NC, N_PAD, D), jnp.float32),
        mesh=_mesh,
        scratch_types=[
            pltpu.VMEM((3, NBUF, CHUNK), jnp.int32),
            pltpu.VMEM((3, NBUF, CHUNK), jnp.int32),
            pltpu.VMEM((NBUF, CHUNK, D), jnp.float32),
            pltpu.VMEM_SHARED((N_PAD, D), jnp.float32),
            pltpu.SemaphoreType.DMA((NBUF,)),
            pltpu.SemaphoreType.DMA,
            pltpu.SemaphoreType.DMA,
        ],
    )(y, e4)


def _out_body(p_ref, deg_ref, b_ref, o_ref):
    agg = p_ref[0] + p_ref[1]
    d = deg_ref[0, 1, 0, 0, :] + deg_ref[1, 1, 0, 0, :]
    norm = lax.rsqrt(jnp.clip(d, 1.0, None))
    o_ref[...] = jnp.maximum(agg * norm[:, None] + b_ref[...], 0.0)


def _out_kernel(p, degp, b2):
    return pl.pallas_call(
        _out_body,
        grid=(N_NODES // ROWB,),
        in_specs=[
            pl.BlockSpec((NC, ROWB, D), lambda i: (0, i, 0)),
            pl.BlockSpec((NC, 2, 1, 1, ROWB), lambda i: (0, 0, i, 0, 0)),
            pl.BlockSpec((1, D), lambda i: (0, 0)),
        ],
        out_specs=pl.BlockSpec((ROWB, D), lambda i: (i, 0)),
        out_shape=jax.ShapeDtypeStruct((N_NODES, D), jnp.float32),
    )(p, degp, b2)


def kernel(x, edge_index, W, b):
    ew = edge_index.reshape(2, NC * NS, E_PER_W)
    e4 = ew[:, :, :DNCH * DCH].reshape(2, NC * NS, DNCH, DCH)
    et = ew[:, :, DNCH * DCH:]
    e5 = edge_index.reshape(2, NC * NS, NGRP, NBUF, CHUNK)
    degp = _deg_kernel(e4, et)
    degt = degp[:, :, :N_NODES].reshape(NC, 2, N_NODES // ROWB, 1, ROWB)
    y = _scale_kernel(x, W, degt)
    p = _edge_kernel(y, e5)
    return _out_kernel(p, degt, b.reshape(1, D))
